# baseline (speedup 1.0000x reference)
import functools

import jax
import jax.numpy as jnp
from jax import lax
from jax.experimental import pallas as pl
from jax.experimental.pallas import tpu as pltpu
from jax.experimental.pallas import tpu_sc as plsc

NN = 10000
EE = 320000
D = 128
DH = D // 2
DW = 16
NC = 2
NS = 16
NW = NC * NS
EW = EE // NW
KA, CA = 80, 125
NB = 8
ND = 16
NPAD = 10112
RPT = NPAD // NS
BLK = 1000

_mesh = plsc.VectorSubcoreMesh(core_axis_name="c", subcore_axis_name="s")


@functools.partial(
    pl.kernel,
    out_type=jax.ShapeDtypeStruct((NC, NPAD, DW), jnp.float32),
    mesh=_mesh,
    scratch_types=[
        pltpu.VMEM((KA, CA), jnp.int32),
        pltpu.VMEM((CA, DW), jnp.float32),
        pltpu.VMEM((RPT, DW), jnp.float32),
        pltpu.VMEM_SHARED((NPAD, DW), jnp.float32),
        pltpu.SemaphoreType.DMA,
    ],
    compiler_params=pltpu.CompilerParams(use_tc_tiling_on_sc=False),
)
def _sc_deg(dst_hbm, out_hbm, dst_v, ones_v, zb_v, acc, sem):
    cid = lax.axis_index("c")
    sid = lax.axis_index("s")
    wid = sid * NC + cid

    def fill(i, carry):
        ones_v.at[i][pl.ds(0, DW)] = jnp.ones((DW,), jnp.float32)
        return carry

    lax.fori_loop(0, CA, fill, 0)

    def zfill(i, carry):
        zb_v.at[i][pl.ds(0, DW)] = jnp.zeros((DW,), jnp.float32)
        return carry

    lax.fori_loop(0, RPT, zfill, 0)

    pltpu.sync_copy(dst_hbm.at[wid], dst_v)
    pltpu.sync_copy(zb_v, acc.at[pl.ds(sid * RPT, RPT)])
    plsc.subcore_barrier()

    for j in range(ND):
        pltpu.async_copy(ones_v, acc.at[dst_v.at[j]], sem, add=True)

    def body(j, carry):
        pltpu.make_async_copy(ones_v, acc.at[dst_v.at[0]], sem).wait()
        pltpu.async_copy(ones_v, acc.at[dst_v.at[j + ND]], sem, add=True)
        return carry

    lax.fori_loop(0, KA - ND, body, 0)
    for j in range(ND):
        pltpu.make_async_copy(ones_v, acc.at[dst_v.at[0]], sem).wait()

    plsc.subcore_barrier()
    pltpu.sync_copy(
        acc.at[pl.ds(sid * RPT, RPT)], out_hbm.at[cid].at[pl.ds(sid * RPT, RPT)]
    )


@functools.partial(
    pl.kernel,
    out_type=jax.ShapeDtypeStruct((2, NC, NPAD, DH), jnp.float32),
    mesh=_mesh,
    scratch_types=[
        pltpu.VMEM((KA, CA), jnp.int32),
        pltpu.VMEM((KA, CA), jnp.int32),
        pltpu.VMEM((NB, CA, DH), jnp.float32),
        pltpu.VMEM((80, DH), jnp.float32),
        pltpu.VMEM_SHARED((NPAD, DH), jnp.float32),
        [pltpu.SemaphoreType.DMA] * NB,
        [pltpu.SemaphoreType.DMA] * NB,
    ],
    compiler_params=pltpu.CompilerParams(use_tc_tiling_on_sc=False),
)
def _sc_agg(hsa_hbm, hsb_hbm, src_hbm, dst_hbm, out_hbm,
            src_v, dst_v, gb, zb, acc, gsem, ssem):
    cid = lax.axis_index("c")
    sid = lax.axis_index("s")
    wid = sid * NC + cid
    base = sid * RPT

    def zfill(i, carry):
        row = zb.at[i]
        for l in range(DH // 16):
            row[pl.ds(l * 16, 16)] = jnp.zeros((16,), jnp.float32)
        return carry

    lax.fori_loop(0, 80, zfill, 0)

    pltpu.sync_copy(src_hbm.at[wid], src_v)
    pltpu.sync_copy(dst_hbm.at[wid], dst_v)

    for h, hs_hbm in enumerate((hsa_hbm, hsb_hbm)):
        for k in range(RPT // 80):
            pltpu.sync_copy(zb, acc.at[pl.ds(base + k * 80, 80)])
        if RPT % 80:
            pltpu.sync_copy(
                zb.at[pl.ds(0, RPT % 80)],
                acc.at[pl.ds(base + RPT - RPT % 80, RPT % 80)],
            )
        plsc.subcore_barrier()

        for b in range(NB):
            pltpu.async_copy(hs_hbm.at[src_v.at[b]], gb.at[b], gsem[b])

        def body(g, carry):
            for b in range(NB):
                j = NB * g + b
                pltpu.make_async_copy(hs_hbm.at[src_v.at[0]], gb.at[b], gsem[b]).wait()
                pltpu.async_copy(gb.at[b], acc.at[dst_v.at[j]], ssem[b], add=True)

                @pl.when(j + NB < KA)
                def _():
                    pltpu.make_async_copy(gb.at[b], acc.at[dst_v.at[0]], ssem[b]).wait()
                    pltpu.async_copy(hs_hbm.at[src_v.at[j + NB]], gb.at[b], gsem[b])

            return carry

        lax.fori_loop(0, KA // NB, body, 0)
        for b in range(NB):
            pltpu.make_async_copy(gb.at[b], acc.at[dst_v.at[0]], ssem[b]).wait()

        plsc.subcore_barrier()
        pltpu.sync_copy(
            acc.at[pl.ds(base, RPT)],
            out_hbm.at[h].at[cid].at[pl.ds(base, RPT)],
        )
        if h == 0:
            plsc.subcore_barrier()


def _deg_inv_sqrt(degp_ref):
    deg = degp_ref[0, :, 0] + degp_ref[1, :, 0] + 1.0
    return lax.rsqrt(deg)[:, None]


def _agg_full(acc_ref):
    return jnp.concatenate(
        [acc_ref[0, 0] + acc_ref[0, 1], acc_ref[1, 0] + acc_ref[1, 1]], axis=1
    )


def _tc1_body(x_ref, w_ref, degp_ref, hsa_ref, hsb_ref):
    d = _deg_inv_sqrt(degp_ref)
    hs = jnp.dot(x_ref[...], w_ref[...], preferred_element_type=jnp.float32) * d
    hsa_ref[...] = hs[:, :DH]
    hsb_ref[...] = hs[:, DH:]


def _tc2_body(acc_ref, hsa_ref, hsb_ref, degp_ref, w_ref, b_ref, a_ref,
              oa_ref, ob_ref):
    d = _deg_inv_sqrt(degp_ref)
    hs = jnp.concatenate([hsa_ref[...], hsb_ref[...]], axis=1)
    pre = (_agg_full(acc_ref) + hs) * d + b_ref[...]
    a = a_ref[0, 0]
    h1 = jnp.where(pre >= 0.0, pre, a * pre)
    hs2 = jnp.dot(h1, w_ref[...], preferred_element_type=jnp.float32) * d
    oa_ref[...] = hs2[:, :DH]
    ob_ref[...] = hs2[:, DH:]


def _tc3_body(acc_ref, hsa_ref, hsb_ref, degp_ref, b_ref, a_ref, out_ref):
    d = _deg_inv_sqrt(degp_ref)
    hs = jnp.concatenate([hsa_ref[...], hsb_ref[...]], axis=1)
    pre = (_agg_full(acc_ref) + hs) * d + b_ref[...]
    a = a_ref[0, 0]
    h2 = jnp.where(pre >= 0.0, pre, a * pre)
    m = jnp.max(h2, axis=1, keepdims=True)
    lse = jnp.log(jnp.sum(jnp.exp(h2 - m), axis=1, keepdims=True)) + m
    out_ref[...] = h2 - lse


_GRID = (NN // BLK,)
_row = pl.BlockSpec((BLK, D), lambda j: (j, 0))
_half = pl.BlockSpec((BLK, DH), lambda j: (j, 0))
_wspec = pl.BlockSpec((D, D), lambda j: (0, 0))
_degspec = pl.BlockSpec((NC, BLK, DW), lambda j: (0, j, 0))
_accspec = pl.BlockSpec((2, NC, BLK, DH), lambda j: (0, 0, j, 0))
_bspec = pl.BlockSpec((1, D), lambda j: (0, 0))
_aspec = pl.BlockSpec((1, 1), lambda j: (0, 0))
_halfout = jax.ShapeDtypeStruct((NN, DH), jnp.float32)

_tc1 = pl.pallas_call(
    _tc1_body, grid=_GRID,
    in_specs=[_row, _wspec, _degspec],
    out_specs=(_half, _half), out_shape=(_halfout, _halfout),
)
_tc2 = pl.pallas_call(
    _tc2_body, grid=_GRID,
    in_specs=[_accspec, _half, _half, _degspec, _wspec, _bspec, _aspec],
    out_specs=(_half, _half), out_shape=(_halfout, _halfout),
)
_tc3 = pl.pallas_call(
    _tc3_body, grid=_GRID,
    in_specs=[_accspec, _half, _half, _degspec, _bspec, _aspec],
    out_specs=_row, out_shape=jax.ShapeDtypeStruct((NN, D), jnp.float32),
)


def kernel(x, edge_index, W1, b1, W2, b2, prelu_a):
    src_a = edge_index[0].reshape(NW, KA, CA)
    dst_a = edge_index[1].reshape(NW, KA, CA)

    degp = _sc_deg(dst_a)

    hsa1, hsb1 = _tc1(x, W1, degp)
    acc1 = _sc_agg(hsa1, hsb1, src_a, dst_a)
    hsa2, hsb2 = _tc2(acc1, hsa1, hsb1, degp, W2,
                      b1.reshape(1, D), prelu_a.reshape(1, 1))
    acc2 = _sc_agg(hsa2, hsb2, src_a, dst_a)
    return _tc3(acc2, hsa2, hsb2, degp, b2.reshape(1, D), prelu_a.reshape(1, 1))


# speedup vs baseline: 1.0055x; 1.0055x over previous
import functools

import jax
import jax.numpy as jnp
from jax import lax
from jax.experimental import pallas as pl
from jax.experimental.pallas import tpu as pltpu
from jax.experimental.pallas import tpu_sc as plsc

NN = 10000
EE = 320000
D = 128
DH = D // 2
DW = 16
NC = 2
NS = 16
NW = NC * NS
EW = EE // NW
KA, CA = 100, 100
NB = 10
ND = 16
NPAD = 10112
RPT = NPAD // NS
BLK = 1000

_mesh = plsc.VectorSubcoreMesh(core_axis_name="c", subcore_axis_name="s")


@functools.partial(
    pl.kernel,
    out_type=jax.ShapeDtypeStruct((NC, NPAD, DW), jnp.float32),
    mesh=_mesh,
    scratch_types=[
        pltpu.VMEM((KA, CA), jnp.int32),
        pltpu.VMEM((CA, DW), jnp.float32),
        pltpu.VMEM((RPT, DW), jnp.float32),
        pltpu.VMEM_SHARED((NPAD, DW), jnp.float32),
        pltpu.SemaphoreType.DMA,
    ],
    compiler_params=pltpu.CompilerParams(use_tc_tiling_on_sc=False),
)
def _sc_deg(dst_hbm, out_hbm, dst_v, ones_v, zb_v, acc, sem):
    cid = lax.axis_index("c")
    sid = lax.axis_index("s")
    wid = sid * NC + cid

    def fill(i, carry):
        ones_v.at[i][pl.ds(0, DW)] = jnp.ones((DW,), jnp.float32)
        return carry

    lax.fori_loop(0, CA, fill, 0)

    def zfill(i, carry):
        zb_v.at[i][pl.ds(0, DW)] = jnp.zeros((DW,), jnp.float32)
        return carry

    lax.fori_loop(0, RPT, zfill, 0)

    pltpu.sync_copy(dst_hbm.at[wid], dst_v)
    pltpu.sync_copy(zb_v, acc.at[pl.ds(sid * RPT, RPT)])
    plsc.subcore_barrier()

    for j in range(ND):
        pltpu.async_copy(ones_v, acc.at[dst_v.at[j]], sem, add=True)

    def body(j, carry):
        pltpu.make_async_copy(ones_v, acc.at[dst_v.at[0]], sem).wait()
        pltpu.async_copy(ones_v, acc.at[dst_v.at[j + ND]], sem, add=True)
        return carry

    lax.fori_loop(0, KA - ND, body, 0)
    for j in range(ND):
        pltpu.make_async_copy(ones_v, acc.at[dst_v.at[0]], sem).wait()

    plsc.subcore_barrier()
    pltpu.sync_copy(
        acc.at[pl.ds(sid * RPT, RPT)], out_hbm.at[cid].at[pl.ds(sid * RPT, RPT)]
    )


@functools.partial(
    pl.kernel,
    out_type=jax.ShapeDtypeStruct((2, NC, NPAD, DH), jnp.float32),
    mesh=_mesh,
    scratch_types=[
        pltpu.VMEM((KA, CA), jnp.int32),
        pltpu.VMEM((KA, CA), jnp.int32),
        pltpu.VMEM((NB, CA, DH), jnp.float32),
        pltpu.VMEM((80, DH), jnp.float32),
        pltpu.VMEM_SHARED((NPAD, DH), jnp.float32),
        [pltpu.SemaphoreType.DMA] * NB,
        [pltpu.SemaphoreType.DMA] * NB,
    ],
    compiler_params=pltpu.CompilerParams(use_tc_tiling_on_sc=False),
)
def _sc_agg(hsa_hbm, hsb_hbm, src_hbm, dst_hbm, out_hbm,
            src_v, dst_v, gb, zb, acc, gsem, ssem):
    cid = lax.axis_index("c")
    sid = lax.axis_index("s")
    wid = sid * NC + cid
    base = sid * RPT

    def zfill(i, carry):
        row = zb.at[i]
        for l in range(DH // 16):
            row[pl.ds(l * 16, 16)] = jnp.zeros((16,), jnp.float32)
        return carry

    lax.fori_loop(0, 80, zfill, 0)

    pltpu.sync_copy(src_hbm.at[wid], src_v)
    pltpu.sync_copy(dst_hbm.at[wid], dst_v)

    for h, hs_hbm in enumerate((hsa_hbm, hsb_hbm)):
        for k in range(RPT // 80):
            pltpu.sync_copy(zb, acc.at[pl.ds(base + k * 80, 80)])
        if RPT % 80:
            pltpu.sync_copy(
                zb.at[pl.ds(0, RPT % 80)],
                acc.at[pl.ds(base + RPT - RPT % 80, RPT % 80)],
            )
        plsc.subcore_barrier()

        for b in range(NB):
            pltpu.async_copy(hs_hbm.at[src_v.at[b]], gb.at[b], gsem[b])

        def body(g, carry):
            for b in range(NB):
                j = NB * g + b
                pltpu.make_async_copy(hs_hbm.at[src_v.at[0]], gb.at[b], gsem[b]).wait()
                pltpu.async_copy(gb.at[b], acc.at[dst_v.at[j]], ssem[b], add=True)

                @pl.when(j + NB < KA)
                def _():
                    pltpu.make_async_copy(gb.at[b], acc.at[dst_v.at[0]], ssem[b]).wait()
                    pltpu.async_copy(hs_hbm.at[src_v.at[j + NB]], gb.at[b], gsem[b])

            return carry

        lax.fori_loop(0, KA // NB, body, 0)
        for b in range(NB):
            pltpu.make_async_copy(gb.at[b], acc.at[dst_v.at[0]], ssem[b]).wait()

        plsc.subcore_barrier()
        pltpu.sync_copy(
            acc.at[pl.ds(base, RPT)],
            out_hbm.at[h].at[cid].at[pl.ds(base, RPT)],
        )
        if h == 0:
            plsc.subcore_barrier()


def _deg_inv_sqrt(degp_ref):
    deg = degp_ref[0, :, 0] + degp_ref[1, :, 0] + 1.0
    return lax.rsqrt(deg)[:, None]


def _agg_full(acc_ref):
    return jnp.concatenate(
        [acc_ref[0, 0] + acc_ref[0, 1], acc_ref[1, 0] + acc_ref[1, 1]], axis=1
    )


def _tc1_body(x_ref, w_ref, degp_ref, hsa_ref, hsb_ref):
    d = _deg_inv_sqrt(degp_ref)
    hs = jnp.dot(x_ref[...], w_ref[...], preferred_element_type=jnp.float32) * d
    hsa_ref[...] = hs[:, :DH]
    hsb_ref[...] = hs[:, DH:]


def _tc2_body(acc_ref, hsa_ref, hsb_ref, degp_ref, w_ref, b_ref, a_ref,
              oa_ref, ob_ref):
    d = _deg_inv_sqrt(degp_ref)
    hs = jnp.concatenate([hsa_ref[...], hsb_ref[...]], axis=1)
    pre = (_agg_full(acc_ref) + hs) * d + b_ref[...]
    a = a_ref[0, 0]
    h1 = jnp.where(pre >= 0.0, pre, a * pre)
    hs2 = jnp.dot(h1, w_ref[...], preferred_element_type=jnp.float32) * d
    oa_ref[...] = hs2[:, :DH]
    ob_ref[...] = hs2[:, DH:]


def _tc3_body(acc_ref, hsa_ref, hsb_ref, degp_ref, b_ref, a_ref, out_ref):
    d = _deg_inv_sqrt(degp_ref)
    hs = jnp.concatenate([hsa_ref[...], hsb_ref[...]], axis=1)
    pre = (_agg_full(acc_ref) + hs) * d + b_ref[...]
    a = a_ref[0, 0]
    h2 = jnp.where(pre >= 0.0, pre, a * pre)
    m = jnp.max(h2, axis=1, keepdims=True)
    lse = jnp.log(jnp.sum(jnp.exp(h2 - m), axis=1, keepdims=True)) + m
    out_ref[...] = h2 - lse


_GRID = (NN // BLK,)
_row = pl.BlockSpec((BLK, D), lambda j: (j, 0))
_half = pl.BlockSpec((BLK, DH), lambda j: (j, 0))
_wspec = pl.BlockSpec((D, D), lambda j: (0, 0))
_degspec = pl.BlockSpec((NC, BLK, DW), lambda j: (0, j, 0))
_accspec = pl.BlockSpec((2, NC, BLK, DH), lambda j: (0, 0, j, 0))
_bspec = pl.BlockSpec((1, D), lambda j: (0, 0))
_aspec = pl.BlockSpec((1, 1), lambda j: (0, 0))
_halfout = jax.ShapeDtypeStruct((NN, DH), jnp.float32)

_tc1 = pl.pallas_call(
    _tc1_body, grid=_GRID,
    in_specs=[_row, _wspec, _degspec],
    out_specs=(_half, _half), out_shape=(_halfout, _halfout),
)
_tc2 = pl.pallas_call(
    _tc2_body, grid=_GRID,
    in_specs=[_accspec, _half, _half, _degspec, _wspec, _bspec, _aspec],
    out_specs=(_half, _half), out_shape=(_halfout, _halfout),
)
_tc3 = pl.pallas_call(
    _tc3_body, grid=_GRID,
    in_specs=[_accspec, _half, _half, _degspec, _bspec, _aspec],
    out_specs=_row, out_shape=jax.ShapeDtypeStruct((NN, D), jnp.float32),
)


def kernel(x, edge_index, W1, b1, W2, b2, prelu_a):
    src_a = edge_index[0].reshape(NW, KA, CA)
    dst_a = edge_index[1].reshape(NW, KA, CA)

    degp = _sc_deg(dst_a)

    hsa1, hsb1 = _tc1(x, W1, degp)
    acc1 = _sc_agg(hsa1, hsb1, src_a, dst_a)
    hsa2, hsb2 = _tc2(acc1, hsa1, hsb1, degp, W2,
                      b1.reshape(1, D), prelu_a.reshape(1, 1))
    acc2 = _sc_agg(hsa2, hsb2, src_a, dst_a)
    return _tc3(acc2, hsa2, hsb2, degp, b2.reshape(1, D), prelu_a.reshape(1, 1))


# speedup vs baseline: 1.1032x; 1.0972x over previous
import functools

import jax
import jax.numpy as jnp
from jax import lax
from jax.experimental import pallas as pl
from jax.experimental.pallas import tpu as pltpu
from jax.experimental.pallas import tpu_sc as plsc

NN = 10000
EE = 320000
D = 128
DH = D // 2
DW = 16
NC = 2
NS = 16
NW = NC * NS
EW = EE // NW
KD, CD = 100, 100
KB, CB = 200, 100
NB = 5
ND = 16
NPAD = 10112
RPT = NPAD // NS
BLK = 1000

_mesh = plsc.VectorSubcoreMesh(core_axis_name="c", subcore_axis_name="s")


@functools.partial(
    pl.kernel,
    out_type=jax.ShapeDtypeStruct((NC, NPAD, DW), jnp.float32),
    mesh=_mesh,
    scratch_types=[
        pltpu.VMEM((KD, CD), jnp.int32),
        pltpu.VMEM((CD, DW), jnp.float32),
        pltpu.VMEM((RPT, DW), jnp.float32),
        pltpu.VMEM_SHARED((NPAD, DW), jnp.float32),
        pltpu.SemaphoreType.DMA,
    ],
    compiler_params=pltpu.CompilerParams(use_tc_tiling_on_sc=False),
)
def _sc_deg(dst_hbm, out_hbm, dst_v, ones_v, zb_v, acc, sem):
    cid = lax.axis_index("c")
    sid = lax.axis_index("s")
    wid = sid * NC + cid

    def fill(i, carry):
        ones_v.at[i][pl.ds(0, DW)] = jnp.ones((DW,), jnp.float32)
        return carry

    lax.fori_loop(0, CD, fill, 0)

    def zfill(i, carry):
        zb_v.at[i][pl.ds(0, DW)] = jnp.zeros((DW,), jnp.float32)
        return carry

    lax.fori_loop(0, RPT, zfill, 0)

    pltpu.sync_copy(dst_hbm.at[wid], dst_v)
    pltpu.sync_copy(zb_v, acc.at[pl.ds(sid * RPT, RPT)])
    plsc.subcore_barrier()

    for j in range(ND):
        pltpu.async_copy(ones_v, acc.at[dst_v.at[j]], sem, add=True)

    def body(j, carry):
        pltpu.make_async_copy(ones_v, acc.at[dst_v.at[0]], sem).wait()
        pltpu.async_copy(ones_v, acc.at[dst_v.at[j + ND]], sem, add=True)
        return carry

    lax.fori_loop(0, KD - ND, body, 0)
    for j in range(ND):
        pltpu.make_async_copy(ones_v, acc.at[dst_v.at[0]], sem).wait()

    plsc.subcore_barrier()
    pltpu.sync_copy(
        acc.at[pl.ds(sid * RPT, RPT)], out_hbm.at[cid].at[pl.ds(sid * RPT, RPT)]
    )


@functools.partial(
    pl.kernel,
    out_type=jax.ShapeDtypeStruct((NC, NPAD, DH), jnp.float32),
    mesh=_mesh,
    scratch_types=[
        pltpu.VMEM((KB, CB), jnp.int32),
        pltpu.VMEM((KB, CB), jnp.int32),
        pltpu.VMEM((NB, CB, DH), jnp.float32),
        pltpu.VMEM((80, DH), jnp.float32),
        pltpu.VMEM_SHARED((NPAD, DH), jnp.float32),
        [pltpu.SemaphoreType.DMA] * NB,
        [pltpu.SemaphoreType.DMA] * NB,
    ],
    compiler_params=pltpu.CompilerParams(use_tc_tiling_on_sc=False),
)
def _sc_agg(hs_hbm, src_hbm, dst_hbm, out_hbm,
            src_v, dst_v, gb, zb, acc, gsem, ssem):
    cid = lax.axis_index("c")
    sid = lax.axis_index("s")
    base = sid * RPT
    hsc = hs_hbm.at[cid]

    def zfill(i, carry):
        row = zb.at[i]
        for l in range(DH // 16):
            row[pl.ds(l * 16, 16)] = jnp.zeros((16,), jnp.float32)
        return carry

    lax.fori_loop(0, 80, zfill, 0)

    pltpu.sync_copy(src_hbm.at[sid], src_v)
    pltpu.sync_copy(dst_hbm.at[sid], dst_v)

    for k in range(RPT // 80):
        pltpu.sync_copy(zb, acc.at[pl.ds(base + k * 80, 80)])
    if RPT % 80:
        pltpu.sync_copy(
            zb.at[pl.ds(0, RPT % 80)],
            acc.at[pl.ds(base + RPT - RPT % 80, RPT % 80)],
        )
    plsc.subcore_barrier()

    for b in range(NB):
        pltpu.async_copy(hsc.at[src_v.at[b]], gb.at[b], gsem[b])

    def body(g, carry):
        for b in range(NB):
            j = NB * g + b
            pltpu.make_async_copy(hsc.at[src_v.at[0]], gb.at[b], gsem[b]).wait()
            pltpu.async_copy(gb.at[b], acc.at[dst_v.at[j]], ssem[b], add=True)

            @pl.when(j + NB < KB)
            def _():
                pltpu.make_async_copy(gb.at[b], acc.at[dst_v.at[0]], ssem[b]).wait()
                pltpu.async_copy(hsc.at[src_v.at[j + NB]], gb.at[b], gsem[b])

        return carry

    lax.fori_loop(0, KB // NB, body, 0)
    for b in range(NB):
        pltpu.make_async_copy(gb.at[b], acc.at[dst_v.at[0]], ssem[b]).wait()

    plsc.subcore_barrier()
    pltpu.sync_copy(
        acc.at[pl.ds(base, RPT)],
        out_hbm.at[cid].at[pl.ds(base, RPT)],
    )


def _deg_inv_sqrt(degp_ref):
    deg = degp_ref[0, :, 0] + degp_ref[1, :, 0] + 1.0
    return lax.rsqrt(deg)[:, None]


def _stacked_full(ref):
    return jnp.concatenate([ref[0], ref[1]], axis=1)


def _tc1_body(x_ref, w_ref, degp_ref, hs_ref):
    d = _deg_inv_sqrt(degp_ref)
    hs = jnp.dot(x_ref[...], w_ref[...], preferred_element_type=jnp.float32) * d
    hs_ref[0] = hs[:, :DH]
    hs_ref[1] = hs[:, DH:]


def _tc2_body(acc_ref, hs_ref, degp_ref, w_ref, b_ref, a_ref, out_ref):
    d = _deg_inv_sqrt(degp_ref)
    pre = (_stacked_full(acc_ref) + _stacked_full(hs_ref)) * d + b_ref[...]
    a = a_ref[0, 0]
    h1 = jnp.where(pre >= 0.0, pre, a * pre)
    hs2 = jnp.dot(h1, w_ref[...], preferred_element_type=jnp.float32) * d
    out_ref[0] = hs2[:, :DH]
    out_ref[1] = hs2[:, DH:]


def _tc3_body(acc_ref, hs_ref, degp_ref, b_ref, a_ref, out_ref):
    d = _deg_inv_sqrt(degp_ref)
    pre = (_stacked_full(acc_ref) + _stacked_full(hs_ref)) * d + b_ref[...]
    a = a_ref[0, 0]
    h2 = jnp.where(pre >= 0.0, pre, a * pre)
    m = jnp.max(h2, axis=1, keepdims=True)
    lse = jnp.log(jnp.sum(jnp.exp(h2 - m), axis=1, keepdims=True)) + m
    out_ref[...] = h2 - lse


_GRID = (NN // BLK,)
_row = pl.BlockSpec((BLK, D), lambda j: (j, 0))
_hspec = pl.BlockSpec((2, BLK, DH), lambda j: (0, j, 0))
_wspec = pl.BlockSpec((D, D), lambda j: (0, 0))
_degspec = pl.BlockSpec((NC, BLK, DW), lambda j: (0, j, 0))
_bspec = pl.BlockSpec((1, D), lambda j: (0, 0))
_aspec = pl.BlockSpec((1, 1), lambda j: (0, 0))
_hsout = jax.ShapeDtypeStruct((2, NN, DH), jnp.float32)

_tc1 = pl.pallas_call(
    _tc1_body, grid=_GRID,
    in_specs=[_row, _wspec, _degspec],
    out_specs=_hspec, out_shape=_hsout,
)
_tc2 = pl.pallas_call(
    _tc2_body, grid=_GRID,
    in_specs=[_hspec, _hspec, _degspec, _wspec, _bspec, _aspec],
    out_specs=_hspec, out_shape=_hsout,
)
_tc3 = pl.pallas_call(
    _tc3_body, grid=_GRID,
    in_specs=[_hspec, _hspec, _degspec, _bspec, _aspec],
    out_specs=_row, out_shape=jax.ShapeDtypeStruct((NN, D), jnp.float32),
)


def kernel(x, edge_index, W1, b1, W2, b2, prelu_a):
    src_d = edge_index[0].reshape(NW, KD, CD)
    dst_d = edge_index[1].reshape(NW, KD, CD)
    src_s = edge_index[0].reshape(NS, KB, CB)
    dst_s = edge_index[1].reshape(NS, KB, CB)
    del src_d

    degp = _sc_deg(dst_d)

    hs1 = _tc1(x, W1, degp)
    acc1 = _sc_agg(hs1, src_s, dst_s)
    hs2 = _tc2(acc1, hs1, degp, W2, b1.reshape(1, D), prelu_a.reshape(1, 1))
    acc2 = _sc_agg(hs2, src_s, dst_s)
    return _tc3(acc2, hs2, degp, b2.reshape(1, D), prelu_a.reshape(1, 1))


# speedup vs baseline: 1.1044x; 1.0011x over previous
import functools

import jax
import jax.numpy as jnp
from jax import lax
from jax.experimental import pallas as pl
from jax.experimental.pallas import tpu as pltpu
from jax.experimental.pallas import tpu_sc as plsc

NN = 10000
EE = 320000
D = 128
DH = D // 2
DW = 16
NC = 2
NS = 16
NW = NC * NS
KD, CD = 100, 100
KB, CB = 200, 100
NB = 5
ND = 16
NPAD = 10112
RPT = NPAD // NS
BLK = 1000

_mesh = plsc.VectorSubcoreMesh(core_axis_name="c", subcore_axis_name="s")


@functools.partial(
    pl.kernel,
    out_type=jax.ShapeDtypeStruct((NC, NPAD, DW), jnp.float32),
    mesh=_mesh,
    scratch_types=[
        pltpu.VMEM((KD, CD), jnp.int32),
        pltpu.VMEM((CD, DW), jnp.float32),
        pltpu.VMEM((RPT, DW), jnp.float32),
        pltpu.VMEM_SHARED((NPAD, DW), jnp.float32),
        pltpu.SemaphoreType.DMA,
    ],
    compiler_params=pltpu.CompilerParams(use_tc_tiling_on_sc=False),
)
def _sc_deg(dst_hbm, out_hbm, dst_v, ones_v, zb_v, acc, sem):
    cid = lax.axis_index("c")
    sid = lax.axis_index("s")
    wid = sid * NC + cid

    def fill(i, carry):
        ones_v.at[i][pl.ds(0, DW)] = jnp.ones((DW,), jnp.float32)
        return carry

    lax.fori_loop(0, CD, fill, 0)

    def zfill(i, carry):
        zb_v.at[i][pl.ds(0, DW)] = jnp.zeros((DW,), jnp.float32)
        return carry

    lax.fori_loop(0, RPT, zfill, 0)

    pltpu.sync_copy(dst_hbm.at[wid], dst_v)
    pltpu.sync_copy(zb_v, acc.at[pl.ds(sid * RPT, RPT)])
    plsc.subcore_barrier()

    for j in range(ND):
        pltpu.async_copy(ones_v, acc.at[dst_v.at[j]], sem, add=True)

    def body(j, carry):
        pltpu.make_async_copy(ones_v, acc.at[dst_v.at[0]], sem).wait()
        pltpu.async_copy(ones_v, acc.at[dst_v.at[j + ND]], sem, add=True)
        return carry

    lax.fori_loop(0, KD - ND, body, 0)
    for j in range(ND):
        pltpu.make_async_copy(ones_v, acc.at[dst_v.at[0]], sem).wait()

    plsc.subcore_barrier()
    pltpu.sync_copy(
        acc.at[pl.ds(sid * RPT, RPT)], out_hbm.at[cid].at[pl.ds(sid * RPT, RPT)]
    )


@functools.partial(
    pl.kernel,
    out_type=jax.ShapeDtypeStruct((NC, NPAD, DH), jnp.float32),
    mesh=_mesh,
    scratch_types=[
        pltpu.VMEM((KB, CB), jnp.int32),
        pltpu.VMEM((KB, CB), jnp.int32),
        pltpu.VMEM((NB, CB, DH), jnp.float32),
        pltpu.VMEM((80, DH), jnp.float32),
        pltpu.VMEM_SHARED((NPAD, DH), jnp.float32),
        [pltpu.SemaphoreType.DMA] * NB,
        [pltpu.SemaphoreType.DMA] * NB,
    ],
    compiler_params=pltpu.CompilerParams(use_tc_tiling_on_sc=False),
)
def _sc_agg(hs_hbm, src_hbm, dst_hbm, out_hbm,
            src_v, dst_v, gb, zb, acc, gsem, ssem):
    cid = lax.axis_index("c")
    sid = lax.axis_index("s")
    base = sid * RPT
    hsc = hs_hbm.at[cid]

    def zfill(i, carry):
        row = zb.at[i]
        for l in range(DH // 16):
            row[pl.ds(l * 16, 16)] = jnp.zeros((16,), jnp.float32)
        return carry

    lax.fori_loop(0, 80, zfill, 0)

    pltpu.sync_copy(src_hbm.at[sid], src_v)
    pltpu.sync_copy(dst_hbm.at[sid], dst_v)

    for k in range(RPT // 80):
        pltpu.sync_copy(zb, acc.at[pl.ds(base + k * 80, 80)])
    if RPT % 80:
        pltpu.sync_copy(
            zb.at[pl.ds(0, RPT % 80)],
            acc.at[pl.ds(base + RPT - RPT % 80, RPT % 80)],
        )
    plsc.subcore_barrier()

    for b in range(NB):
        pltpu.async_copy(hsc.at[src_v.at[b]], gb.at[b], gsem[b])

    def body(g, carry):
        for b in range(NB):
            j = NB * g + b
            pltpu.make_async_copy(hsc.at[src_v.at[0]], gb.at[b], gsem[b]).wait()
            pltpu.async_copy(gb.at[b], acc.at[dst_v.at[j]], ssem[b], add=True)

            @pl.when(j + NB < KB)
            def _():
                pltpu.make_async_copy(gb.at[b], acc.at[dst_v.at[0]], ssem[b]).wait()
                pltpu.async_copy(hsc.at[src_v.at[j + NB]], gb.at[b], gsem[b])

        return carry

    lax.fori_loop(0, KB // NB, body, 0)
    for b in range(NB):
        pltpu.make_async_copy(gb.at[b], acc.at[dst_v.at[0]], ssem[b]).wait()

    plsc.subcore_barrier()
    pltpu.sync_copy(
        acc.at[pl.ds(base, RPT)],
        out_hbm.at[cid].at[pl.ds(base, RPT)],
    )


def _deg_inv_sqrt(degp_ref):
    deg = degp_ref[0, :, 0] + degp_ref[1, :, 0] + 1.0
    return lax.rsqrt(deg)[:, None]


def _stacked_full(ref):
    return jnp.concatenate([ref[0], ref[1]], axis=1)


def _tc1_body(x_ref, w_ref, degp_ref, hs_ref):
    d = _deg_inv_sqrt(degp_ref)
    hs = jnp.dot(x_ref[...], w_ref[...], preferred_element_type=jnp.float32) * d
    hs_ref[0] = hs[:, :DH]
    hs_ref[1] = hs[:, DH:]


def _tc2_body(acc_ref, hs_ref, degp_ref, w_ref, b_ref, a_ref, out_ref):
    d = _deg_inv_sqrt(degp_ref)
    pre = (_stacked_full(acc_ref) + _stacked_full(hs_ref)) * d + b_ref[...]
    a = a_ref[0, 0]
    h1 = jnp.where(pre >= 0.0, pre, a * pre)
    hs2 = jnp.dot(h1, w_ref[...], preferred_element_type=jnp.float32) * d
    out_ref[0] = hs2[:, :DH]
    out_ref[1] = hs2[:, DH:]


def _tc3_body(acc_ref, hs_ref, degp_ref, b_ref, a_ref, out_ref):
    d = _deg_inv_sqrt(degp_ref)
    pre = (_stacked_full(acc_ref) + _stacked_full(hs_ref)) * d + b_ref[...]
    a = a_ref[0, 0]
    h2 = jnp.where(pre >= 0.0, pre, a * pre)
    m = jnp.max(h2, axis=1, keepdims=True)
    lse = jnp.log(jnp.sum(jnp.exp(h2 - m), axis=1, keepdims=True)) + m
    out_ref[...] = h2 - lse


_GRID = (NN // BLK,)
_row = pl.BlockSpec((BLK, D), lambda j: (j, 0))
_hspec = pl.BlockSpec((2, BLK, DH), lambda j: (0, j, 0))
_wspec = pl.BlockSpec((D, D), lambda j: (0, 0))
_degspec = pl.BlockSpec((NC, BLK, DW), lambda j: (0, j, 0))
_bspec = pl.BlockSpec((1, D), lambda j: (0, 0))
_aspec = pl.BlockSpec((1, 1), lambda j: (0, 0))
_hsout = jax.ShapeDtypeStruct((2, NN, DH), jnp.float32)

_tc1 = pl.pallas_call(
    _tc1_body, grid=_GRID,
    in_specs=[_row, _wspec, _degspec],
    out_specs=_hspec, out_shape=_hsout,
)
_tc2 = pl.pallas_call(
    _tc2_body, grid=_GRID,
    in_specs=[_hspec, _hspec, _degspec, _wspec, _bspec, _aspec],
    out_specs=_hspec, out_shape=_hsout,
)
_tc3 = pl.pallas_call(
    _tc3_body, grid=_GRID,
    in_specs=[_hspec, _hspec, _degspec, _bspec, _aspec],
    out_specs=_row, out_shape=jax.ShapeDtypeStruct((NN, D), jnp.float32),
)


def kernel(x, edge_index, W1, b1, W2, b2, prelu_a):
    dst_d = edge_index[1].reshape(NW, KD, CD)
    src_s = edge_index[0].reshape(NS, KB, CB)
    dst_s = edge_index[1].reshape(NS, KB, CB)

    degp = _sc_deg(dst_d)

    hs1 = _tc1(x, W1, degp)
    acc1 = _sc_agg(hs1, src_s, dst_s)
    hs2 = _tc2(acc1, hs1, degp, W2, b1.reshape(1, D), prelu_a.reshape(1, 1))
    acc2 = _sc_agg(hs2, src_s, dst_s)
    return _tc3(acc2, hs2, degp, b2.reshape(1, D), prelu_a.reshape(1, 1))
